# baseline (speedup 1.0000x reference)
import functools

import jax
import jax.numpy as jnp
from jax import lax
from jax.experimental import pallas as pl
from jax.experimental.pallas import tpu as pltpu
from jax.experimental.pallas import tpu_sc as plsc

D = 32
NB = 16384
NJ = 50
NC = 2
NS = 16
NW = NC * NS
BW = NB // NW
NVREG = BW * D // 16

_mesh = plsc.VectorSubcoreMesh(core_axis_name="c", subcore_axis_name="s")

NR = 1000000
TBLK = 768
NBLKF = NR // TBLK
TREM = NR - NBLKF * TBLK
TITER = (NBLKF + NW - 1) // NW


@functools.partial(
    pl.kernel,
    out_type=jax.ShapeDtypeStruct((NR, D), jnp.float32),
    mesh=_mesh,
    compiler_params=pltpu.CompilerParams(
        use_tc_tiling_on_sc=False, needs_layout_passes=False),
    scratch_types=[
        pltpu.VMEM((2, D, TBLK + 1), jnp.float32),
        pltpu.VMEM((2, TBLK, D), jnp.float32),
        pltpu.SemaphoreType.DMA,
        pltpu.SemaphoreType.DMA,
        pltpu.SemaphoreType.DMA,
        pltpu.SemaphoreType.DMA,
    ],
)
def _table_transpose(tabt_hbm, out_hbm, inb, outb, gsem0, gsem1, ssem0, ssem1):
    wid = lax.axis_index("s") * NC + lax.axis_index("c")
    gsems = (gsem0, gsem1)
    ssems = (ssem0, ssem1)
    iota16 = lax.iota(jnp.int32, 16)

    def blk_of(k):
        return wid + k * NW

    def load_start(k, b):
        @pl.when(blk_of(k) < NBLKF)
        def _():
            pltpu.async_copy(
                tabt_hbm.at[:, pl.ds(blk_of(k) * TBLK, TBLK)],
                inb.at[b, :, pl.ds(0, TBLK)], gsems[b])

    def load_wait(k, b):
        @pl.when(blk_of(k) < NBLKF)
        def _():
            pltpu.make_async_copy(
                tabt_hbm.at[:, pl.ds(0, TBLK)],
                inb.at[b, :, pl.ds(0, TBLK)], gsems[b]).wait()

    def store_start(k, b):
        @pl.when(blk_of(k) < NBLKF)
        def _():
            pltpu.async_copy(
                outb.at[b], out_hbm.at[pl.ds(blk_of(k) * TBLK, TBLK)],
                ssems[b])

    def store_wait(k, b):
        @pl.when(blk_of(k) < NBLKF)
        def _():
            pltpu.make_async_copy(
                outb.at[b], out_hbm.at[pl.ds(0, TBLK)], ssems[b]).wait()

    def transpose(k, b):
        @pl.when(blk_of(k) < NBLKF)
        def _():
            @plsc.parallel_loop(0, TBLK, step=1, unroll=8)
            def _(r):
                outb[b, r, pl.ds(0, 16)] = plsc.load_gather(
                    inb, [jnp.full((16,), b, jnp.int32), iota16,
                          jnp.zeros((16,), jnp.int32) + r])
                outb[b, r, pl.ds(16, 16)] = plsc.load_gather(
                    inb, [jnp.full((16,), b, jnp.int32), iota16 + 16,
                          jnp.zeros((16,), jnp.int32) + r])

    load_start(0, 0)
    load_start(1, 1)
    load_wait(0, 0)
    transpose(0, 0)
    load_start(2, 0)
    store_start(0, 0)
    load_wait(1, 1)
    transpose(1, 1)
    load_start(3, 1)
    store_start(1, 1)

    def steady(t, carry):
        for u in range(2):
            k = 2 + t * 2 + u
            b = u
            load_wait(k, b)
            store_wait(k, b)
            transpose(k, b)
            load_start(k + 2, b)
            store_start(k, b)
        return carry
    lax.fori_loop(0, 20, steady, 0)

    store_wait(TITER - 2, 1)
    store_wait(TITER - 1, 0)

    @pl.when(wid == 0)
    def _():
        pltpu.sync_copy(tabt_hbm.at[:, pl.ds(NBLKF * TBLK, TREM)],
                        inb.at[0, :, pl.ds(0, TREM)])

        @plsc.parallel_loop(0, TREM, step=1, unroll=8)
        def _(r):
            outb[0, r, pl.ds(0, 16)] = plsc.load_gather(
                inb, [jnp.zeros((16,), jnp.int32), iota16,
                      jnp.zeros((16,), jnp.int32) + r])
            outb[0, r, pl.ds(16, 16)] = plsc.load_gather(
                inb, [jnp.zeros((16,), jnp.int32), iota16 + 16,
                      jnp.zeros((16,), jnp.int32) + r])
        pltpu.sync_copy(outb.at[0, pl.ds(0, TREM)],
                        out_hbm.at[pl.ds(NBLKF * TBLK, TREM)])


@functools.partial(
    pl.kernel,
    out_type=[
        jax.ShapeDtypeStruct((NJ, D, NB), jnp.float32),
        jax.ShapeDtypeStruct((NW, 16), jnp.float32),
    ],
    mesh=_mesh,
    compiler_params=pltpu.CompilerParams(
        use_tc_tiling_on_sc=False, needs_layout_passes=False),
    scratch_types=[
        pltpu.VMEM((NJ, BW), jnp.int32),
        pltpu.VMEM((2, BW, D), jnp.float32),
        pltpu.VMEM((2, D, BW + 1), jnp.float32),
        pltpu.VMEM((16,), jnp.float32),
        pltpu.SemaphoreType.DMA,
        pltpu.SemaphoreType.DMA,
        pltpu.SemaphoreType.DMA,
        pltpu.SemaphoreType.DMA,
    ],
)
def _embedding_gather(table_hbm, idx_hbm, out_hbm, psum_hbm,
                      idx_v, rows_v, trans_v, acc_v,
                      gsem0, gsem1, ssem0, ssem1):
    wid = lax.axis_index("s") * NC + lax.axis_index("c")
    b_base = wid * BW

    pltpu.sync_copy(idx_hbm.at[:, pl.ds(b_base, BW)], idx_v)

    gsems = (gsem0, gsem1)
    ssems = (ssem0, ssem1)
    iota16 = lax.iota(jnp.int32, 16)
    zeros_i = jnp.zeros((16,), jnp.int32)
    zeros_f = jnp.zeros((16,), jnp.float32)

    def gather_start(j, b):
        return pltpu.async_copy(
            table_hbm.at[idx_v.at[j]], rows_v.at[b], gsems[b])

    def gather_wait(b):
        pltpu.make_async_copy(
            table_hbm.at[pl.ds(0, BW)], rows_v.at[b], gsems[b]).wait()

    def store_start(j, b):
        return pltpu.async_copy(
            trans_v.at[b, :, pl.ds(0, BW)],
            out_hbm.at[j, :, pl.ds(b_base, BW)], ssems[b])

    def store_wait(b):
        pltpu.make_async_copy(
            trans_v.at[b, :, pl.ds(0, BW)],
            out_hbm.at[0, :, pl.ds(b_base, BW)], ssems[b]).wait()

    bvec = (jnp.full((16,), 0, jnp.int32), jnp.full((16,), 1, jnp.int32))
    iota_hi = iota16 + 16

    def transpose_acc(b, accs):
        @plsc.parallel_loop(0, BW, step=1, unroll=8, carry=accs)
        def new_accs(r, c):
            a0, a1 = c
            rvec = zeros_i + r
            v0 = rows_v[b, r, pl.ds(0, 16)]
            v1 = rows_v[b, r, pl.ds(16, 16)]
            plsc.store_scatter(trans_v, [bvec[b], iota16, rvec], v0)
            plsc.store_scatter(trans_v, [bvec[b], iota_hi, rvec], v1)
            a0 = a0 + v0
            a1 = a1 + v1
            return (a0, a1)
        return new_accs

    accs = (zeros_f, zeros_f)

    gather_start(0, 0)
    gather_wait(0)
    gather_start(1, 1)
    accs = transpose_acc(0, accs)
    store_start(0, 0)
    gather_wait(1)
    gather_start(2, 0)
    accs = transpose_acc(1, accs)
    store_start(1, 1)
    gather_wait(0)
    gather_start(3, 1)
    store_wait(0)
    accs = transpose_acc(0, accs)
    store_start(2, 0)

    def steady(t, accs):
        for k in range(2):
            j = 3 + t * 2 + k
            b = 1 - k
            gather_wait(b)
            gather_start(j + 1, 1 - b)
            store_wait(b)
            accs = transpose_acc(b, accs)
            store_start(j, b)
        return accs
    accs = lax.fori_loop(0, (NJ - 4) // 2, steady, accs)

    gather_wait(1)
    store_wait(1)
    accs = transpose_acc(1, accs)
    store_start(NJ - 1, 1)

    store_wait(0)
    store_wait(1)
    acc_v[...] = accs[0] + accs[1]
    pltpu.sync_copy(acc_v, psum_hbm.at[wid])


def kernel(table, x):
    idx = x.T.astype(jnp.int32)
    t_rm = _table_transpose(table.T)
    out_jcb, psum = _embedding_gather(t_rm, idx)
    loss = jnp.sum(psum)
    return (loss, out_jcb.transpose(2, 0, 1))


# speedup vs baseline: 4.0397x; 4.0397x over previous
import functools

import jax
import jax.numpy as jnp
from jax import lax
from jax.experimental import pallas as pl
from jax.experimental.pallas import tpu as pltpu
from jax.experimental.pallas import tpu_sc as plsc

D = 32
NB = 16384
NJ = 50
NC = 2
NS = 16
NW = NC * NS
BW = NB // NW
NVREG = BW * D // 16

_mesh = plsc.VectorSubcoreMesh(core_axis_name="c", subcore_axis_name="s")


@functools.partial(
    pl.kernel,
    out_type=[
        jax.ShapeDtypeStruct((NJ, D, NB), jnp.float32),
        jax.ShapeDtypeStruct((NW, 16), jnp.float32),
    ],
    mesh=_mesh,
    compiler_params=pltpu.CompilerParams(
        use_tc_tiling_on_sc=False, needs_layout_passes=False),
    scratch_types=[
        pltpu.VMEM((NJ, BW), jnp.int32),
        pltpu.VMEM((2, BW, D), jnp.float32),
        pltpu.VMEM((2, D, BW + 1), jnp.float32),
        pltpu.VMEM((16,), jnp.float32),
        pltpu.SemaphoreType.DMA,
        pltpu.SemaphoreType.DMA,
        pltpu.SemaphoreType.DMA,
        pltpu.SemaphoreType.DMA,
    ],
)
def _embedding_gather(table_hbm, idx_hbm, out_hbm, psum_hbm,
                      idx_v, rows_v, trans_v, acc_v,
                      gsem0, gsem1, ssem0, ssem1):
    wid = lax.axis_index("s") * NC + lax.axis_index("c")
    b_base = wid * BW

    pltpu.sync_copy(idx_hbm.at[:, pl.ds(b_base, BW)], idx_v)

    gsems = (gsem0, gsem1)
    ssems = (ssem0, ssem1)
    iota16 = lax.iota(jnp.int32, 16)
    zeros_i = jnp.zeros((16,), jnp.int32)
    zeros_f = jnp.zeros((16,), jnp.float32)

    def gather_start(j, b):
        return pltpu.async_copy(
            table_hbm.at[idx_v.at[j]], rows_v.at[b], gsems[b])

    def gather_wait(b):
        pltpu.make_async_copy(
            table_hbm.at[pl.ds(0, BW)], rows_v.at[b], gsems[b]).wait()

    def store_start(j, b):
        return pltpu.async_copy(
            trans_v.at[b, :, pl.ds(0, BW)],
            out_hbm.at[j, :, pl.ds(b_base, BW)], ssems[b])

    def store_wait(b):
        pltpu.make_async_copy(
            trans_v.at[b, :, pl.ds(0, BW)],
            out_hbm.at[0, :, pl.ds(b_base, BW)], ssems[b]).wait()

    bvec = (jnp.full((16,), 0, jnp.int32), jnp.full((16,), 1, jnp.int32))
    iota_hi = iota16 + 16

    def transpose_acc(b, accs):
        @plsc.parallel_loop(0, BW, step=1, unroll=8, carry=accs)
        def new_accs(r, c):
            a0, a1 = c
            rvec = zeros_i + r
            v0 = rows_v[b, r, pl.ds(0, 16)]
            v1 = rows_v[b, r, pl.ds(16, 16)]
            plsc.store_scatter(trans_v, [bvec[b], iota16, rvec], v0)
            plsc.store_scatter(trans_v, [bvec[b], iota_hi, rvec], v1)
            a0 = a0 + v0
            a1 = a1 + v1
            return (a0, a1)
        return new_accs

    accs = (zeros_f, zeros_f)

    gather_start(0, 0)
    gather_wait(0)
    gather_start(1, 1)
    accs = transpose_acc(0, accs)
    store_start(0, 0)
    gather_wait(1)
    gather_start(2, 0)
    accs = transpose_acc(1, accs)
    store_start(1, 1)
    gather_wait(0)
    gather_start(3, 1)
    store_wait(0)
    accs = transpose_acc(0, accs)
    store_start(2, 0)

    def steady(t, accs):
        for k in range(2):
            j = 3 + t * 2 + k
            b = 1 - k
            gather_wait(b)
            gather_start(j + 1, 1 - b)
            store_wait(b)
            accs = transpose_acc(b, accs)
            store_start(j, b)
        return accs
    accs = lax.fori_loop(0, (NJ - 4) // 2, steady, accs)

    gather_wait(1)
    store_wait(1)
    accs = transpose_acc(1, accs)
    store_start(NJ - 1, 1)

    store_wait(0)
    store_wait(1)
    acc_v[...] = accs[0] + accs[1]
    pltpu.sync_copy(acc_v, psum_hbm.at[wid])


def kernel(table, x):
    idx = x.T.astype(jnp.int32)
    out_jcb, psum = _embedding_gather(table, idx)
    loss = jnp.sum(psum)
    return (loss, out_jcb.transpose(2, 0, 1))
